# baseline (speedup 1.0000x reference)
import functools

import jax
import jax.numpy as jnp
from jax import lax
from jax.experimental import pallas as pl
from jax.experimental.pallas import tpu as pltpu
from jax.experimental.pallas import tpu_sc as plsc

NC = 2
NS = 16
LANES = 16
CHUNK = 128
SUP = 128
HALF = 128


def _split_body(x_ref, lo_ref, hi_ref):
    lo_ref[...] = x_ref[:, :HALF]
    hi_ref[...] = x_ref[:, HALF:]


def _split(x, bn):
    n, d = x.shape
    return pl.pallas_call(
        _split_body,
        grid=(n // bn,),
        in_specs=[pl.BlockSpec((bn, d), lambda i: (i, 0))],
        out_specs=[
            pl.BlockSpec((bn, HALF), lambda i: (i, 0)),
            pl.BlockSpec((bn, HALF), lambda i: (i, 0)),
        ],
        out_shape=[
            jax.ShapeDtypeStruct((n, HALF), jnp.float32),
            jax.ShapeDtypeStruct((n, HALF), jnp.float32),
        ],
    )(x)


def _sc_aggregate(xlo, xhi, ei2, n_nodes):
    n_pad = ((n_nodes + NS * CHUNK - 1) // (NS * CHUNK)) * (NS * CHUNK)
    rows_per_tile = n_pad // NS
    num_sups = ei2.shape[1] // SUP
    sups_per_tile = num_sups // NS

    mesh = plsc.VectorSubcoreMesh(core_axis_name="c", subcore_axis_name="s")

    @functools.partial(
        pl.kernel,
        mesh=mesh,
        out_type=(
            jax.ShapeDtypeStruct((NC, n_pad, HALF), jnp.float32),
            jax.ShapeDtypeStruct((n_pad,), jnp.float32),
        ),
        scratch_types=[
            pltpu.VMEM_SHARED((n_pad, HALF), jnp.float32),
            pltpu.VMEM_SHARED((n_pad,), jnp.float32),
            pltpu.VMEM((1, SUP), jnp.int32),
            pltpu.VMEM((1, SUP), jnp.int32),
            pltpu.VMEM((SUP, HALF), jnp.float32),
            pltpu.VMEM((SUP,), jnp.float32),
            pltpu.SemaphoreType.DMA,
        ],
    )
    def agg_kernel(xlo_hbm, xhi_hbm, edges_hbm, agg_hbm, deg_hbm,
                   agg_s, deg_s, ridx, cidx, msgs, ones, sem):
        c = lax.axis_index("c")
        t = lax.axis_index("s")
        r0 = t * rows_per_tile

        def zrow(r, _):
            def zcol(j, _):
                msgs[r, pl.ds(j * LANES, LANES)] = jnp.zeros(
                    (LANES,), jnp.float32)
                return 0
            return lax.fori_loop(0, HALF // LANES, zcol, 0)
        lax.fori_loop(0, SUP, zrow, 0)

        def zon(j, _):
            ones[pl.ds(j * LANES, LANES)] = jnp.zeros((LANES,), jnp.float32)
            return 0
        lax.fori_loop(0, SUP // LANES, zon, 0)

        for b in range(rows_per_tile // CHUNK):
            pltpu.sync_copy(msgs.at[pl.ds(0, CHUNK)],
                            agg_s.at[pl.ds(r0 + b * CHUNK, CHUNK)])
            pltpu.sync_copy(ones.at[pl.ds(0, CHUNK)],
                            deg_s.at[pl.ds(r0 + b * CHUNK, CHUNK)])

        def son(j, _):
            ones[pl.ds(j * LANES, LANES)] = jnp.ones((LANES,), jnp.float32)
            return 0
        lax.fori_loop(0, SUP // LANES, son, 0)

        plsc.subcore_barrier()

        def edge_loop(table):
            def body(k, _):
                base = (t + k * NS) * SUP
                pltpu.sync_copy(edges_hbm.at[0, pl.ds(base, SUP)], ridx.at[0])
                pltpu.sync_copy(edges_hbm.at[1, pl.ds(base, SUP)], cidx.at[0])
                pltpu.async_copy(table.at[cidx.at[0]], msgs, sem).wait()
                pltpu.sync_copy(msgs, agg_s.at[ridx.at[0]], add=True)
                pltpu.sync_copy(ones, deg_s.at[ridx.at[0]], add=True)
                return 0
            lax.fori_loop(0, sups_per_tile, body, 0)

        @pl.when(c == 0)
        def _():
            edge_loop(xlo_hbm)

        @pl.when(c == 1)
        def _():
            edge_loop(xhi_hbm)

        plsc.subcore_barrier()

        pltpu.sync_copy(agg_s.at[pl.ds(r0, rows_per_tile)],
                        agg_hbm.at[c, pl.ds(r0, rows_per_tile)])

        @pl.when(c == 0)
        def _():
            pltpu.sync_copy(deg_s.at[pl.ds(r0, rows_per_tile)],
                            deg_hbm.at[pl.ds(r0, rows_per_tile)])

    return agg_kernel(xlo, xhi, ei2)


def _fin_body(agg_ref, deg_ref, x_ref, w_ref, b_ref, out_ref):
    d = jnp.maximum(deg_ref[...], 1.0)
    a = jnp.concatenate([agg_ref[0], agg_ref[1]], axis=-1) / d
    lhs = jnp.concatenate([a, x_ref[...]], axis=-1)
    out_ref[...] = jnp.dot(lhs, w_ref[...],
                           preferred_element_type=jnp.float32) + b_ref[...]


def _finalize(agg, deg_col, x, wcat, bias_row, bn):
    n, d_in = x.shape
    d_out = wcat.shape[1]
    grid = n // bn
    return pl.pallas_call(
        _fin_body,
        grid=(grid,),
        in_specs=[
            pl.BlockSpec((NC, bn, HALF), lambda i: (0, i, 0)),
            pl.BlockSpec((bn, 1), lambda i: (i, 0)),
            pl.BlockSpec((bn, d_in), lambda i: (i, 0)),
            pl.BlockSpec((2 * d_in, d_out), lambda i: (0, 0)),
            pl.BlockSpec((1, d_out), lambda i: (0, 0)),
        ],
        out_specs=pl.BlockSpec((bn, d_out), lambda i: (i, 0)),
        out_shape=jax.ShapeDtypeStruct((n, d_out), jnp.float32),
    )(agg, deg_col, x, wcat, bias_row)


def kernel(x, edge_index, weight, root_weight, bias):
    n, _ = x.shape
    e = edge_index.shape[1]
    n_pad = ((n + NS * CHUNK - 1) // (NS * CHUNK)) * (NS * CHUNK)
    span = NS * SUP * 2
    e_pad = ((e + span - 1) // span) * span

    pad = e_pad - e
    if pad:
        pad_block = jnp.concatenate(
            [jnp.full((1, pad), n_pad - 1, jnp.int32),
             jnp.zeros((1, pad), jnp.int32)], axis=0)
        ei = jnp.concatenate([edge_index, pad_block], axis=1)
    else:
        ei = edge_index
    xlo, xhi = _split(x, bn=1000)
    agg, deg = _sc_aggregate(xlo, xhi, ei, n)

    wcat = jnp.concatenate([weight, root_weight], axis=0)
    return _finalize(agg, deg.reshape(-1, 1), x, wcat,
                     bias.reshape(1, -1), bn=1000)


# speedup vs baseline: 1.2553x; 1.2553x over previous
import functools

import jax
import jax.numpy as jnp
from jax import lax
from jax.experimental import pallas as pl
from jax.experimental.pallas import tpu as pltpu
from jax.experimental.pallas import tpu_sc as plsc

NC = 2
NS = 16
LANES = 16
CHUNK = 128
HALF = 128
NBUF = 2


def _mm_body(x_ref, w_ref, b_ref, tlo_ref, thi_ref, root_ref):
    o = jnp.dot(x_ref[...], w_ref[...], preferred_element_type=jnp.float32)
    d = tlo_ref.shape[1]
    tlo_ref[...] = o[:, :d]
    thi_ref[...] = o[:, d:2 * d]
    root_ref[...] = o[:, 2 * d:] + b_ref[...]


def _matmul(x, wcat, bias_row, bn):
    n, d_in = x.shape
    d_out = bias_row.shape[1]
    grid = n // bn
    return pl.pallas_call(
        _mm_body,
        grid=(grid,),
        in_specs=[
            pl.BlockSpec((bn, d_in), lambda i: (i, 0)),
            pl.BlockSpec((d_in, 2 * d_out), lambda i: (0, 0)),
            pl.BlockSpec((1, d_out), lambda i: (0, 0)),
        ],
        out_specs=[
            pl.BlockSpec((bn, HALF), lambda i: (i, 0)),
            pl.BlockSpec((bn, HALF), lambda i: (i, 0)),
            pl.BlockSpec((bn, d_out), lambda i: (i, 0)),
        ],
        out_shape=[
            jax.ShapeDtypeStruct((n, HALF), jnp.float32),
            jax.ShapeDtypeStruct((n, HALF), jnp.float32),
            jax.ShapeDtypeStruct((n, d_out), jnp.float32),
        ],
    )(x, wcat, bias_row)


def _sc_aggregate(tlo, thi, ei2, n_nodes):
    n_pad = ((n_nodes + NS * CHUNK - 1) // (NS * CHUNK)) * (NS * CHUNK)
    rows_per_tile = n_pad // NS
    num_chunks = ei2.shape[1] // CHUNK
    nk = num_chunks // NS

    mesh = plsc.VectorSubcoreMesh(core_axis_name="c", subcore_axis_name="s")

    @functools.partial(
        pl.kernel,
        mesh=mesh,
        out_type=(
            jax.ShapeDtypeStruct((NC, n_pad, HALF), jnp.float32),
            jax.ShapeDtypeStruct((n_pad,), jnp.float32),
        ),
        scratch_types=[
            pltpu.VMEM_SHARED((n_pad, HALF), jnp.float32),
            pltpu.VMEM_SHARED((n_pad,), jnp.float32),
            pltpu.VMEM((NBUF, CHUNK), jnp.int32),
            pltpu.VMEM((NBUF, CHUNK), jnp.int32),
            pltpu.VMEM((NBUF, CHUNK, HALF), jnp.float32),
            pltpu.VMEM((CHUNK,), jnp.float32),
            pltpu.SemaphoreType.DMA,
            pltpu.SemaphoreType.DMA,
        ],
    )
    def agg_kernel(tlo_hbm, thi_hbm, edges_hbm, agg_hbm, deg_hbm,
                   agg_s, deg_s, ridx, cidx, msgs, ones, sem0, sem1):
        c = lax.axis_index("c")
        t = lax.axis_index("s")
        r0 = t * rows_per_tile
        sems = (sem0, sem1)

        def zrow(r, _):
            def zcol(j, _):
                msgs[0, r, pl.ds(j * LANES, LANES)] = jnp.zeros(
                    (LANES,), jnp.float32)
                return 0
            return lax.fori_loop(0, HALF // LANES, zcol, 0)
        lax.fori_loop(0, CHUNK, zrow, 0)

        def zon(j, _):
            ones[pl.ds(j * LANES, LANES)] = jnp.zeros((LANES,), jnp.float32)
            return 0
        lax.fori_loop(0, CHUNK // LANES, zon, 0)

        for b in range(rows_per_tile // CHUNK):
            pltpu.sync_copy(msgs.at[0], agg_s.at[pl.ds(r0 + b * CHUNK, CHUNK)])
            pltpu.sync_copy(ones, deg_s.at[pl.ds(r0 + b * CHUNK, CHUNK)])

        def son(j, _):
            ones[pl.ds(j * LANES, LANES)] = jnp.ones((LANES,), jnp.float32)
            return 0
        lax.fori_loop(0, CHUNK // LANES, son, 0)

        plsc.subcore_barrier()

        def load_and_fire(g, b, tbl_hbm):
            base = (t + g * NS) * CHUNK
            pltpu.sync_copy(edges_hbm.at[0, pl.ds(base, CHUNK)], ridx.at[b])
            pltpu.sync_copy(edges_hbm.at[1, pl.ds(base, CHUNK)], cidx.at[b])
            pltpu.async_copy(tbl_hbm.at[cidx.at[b]], msgs.at[b], sems[b])

        def drain_and_add(b, tbl_hbm):
            pltpu.make_async_copy(
                tbl_hbm.at[cidx.at[b]], msgs.at[b], sems[b]).wait()
            pltpu.sync_copy(msgs.at[b], agg_s.at[ridx.at[b]], add=True)
            pltpu.sync_copy(ones, deg_s.at[ridx.at[b]], add=True)

        def edge_loop(tbl_hbm):
            for b in range(NBUF):
                load_and_fire(b, b, tbl_hbm)

            def body(kk, _):
                for b in range(NBUF):
                    drain_and_add(b, tbl_hbm)
                    load_and_fire(kk * NBUF + b + NBUF, b, tbl_hbm)
                return 0
            lax.fori_loop(0, nk // NBUF - 1, body, 0)

            for b in range(NBUF):
                drain_and_add(b, tbl_hbm)

        @pl.when(c == 0)
        def _():
            edge_loop(tlo_hbm)

        @pl.when(c == 1)
        def _():
            edge_loop(thi_hbm)

        plsc.subcore_barrier()

        pltpu.sync_copy(agg_s.at[pl.ds(r0, rows_per_tile)],
                        agg_hbm.at[c, pl.ds(r0, rows_per_tile)])

        @pl.when(c == 0)
        def _():
            pltpu.sync_copy(deg_s.at[pl.ds(r0, rows_per_tile)],
                            deg_hbm.at[pl.ds(r0, rows_per_tile)])

    return agg_kernel(tlo, thi, ei2)


def _fin_body(agg_ref, deg_ref, root_ref, out_ref):
    d = jnp.maximum(deg_ref[...], 1.0)
    a = jnp.concatenate([agg_ref[0], agg_ref[1]], axis=-1)
    out_ref[...] = a / d + root_ref[...]


def _finalize(agg, deg_col, root, bn):
    n, d_out = root.shape
    grid = n // bn
    return pl.pallas_call(
        _fin_body,
        grid=(grid,),
        in_specs=[
            pl.BlockSpec((NC, bn, HALF), lambda i: (0, i, 0)),
            pl.BlockSpec((bn, 1), lambda i: (i, 0)),
            pl.BlockSpec((bn, d_out), lambda i: (i, 0)),
        ],
        out_specs=pl.BlockSpec((bn, d_out), lambda i: (i, 0)),
        out_shape=jax.ShapeDtypeStruct((n, d_out), jnp.float32),
    )(agg, deg_col, root)


def kernel(x, edge_index, weight, root_weight, bias):
    n, _ = x.shape
    e = edge_index.shape[1]
    n_pad = ((n + NS * CHUNK - 1) // (NS * CHUNK)) * (NS * CHUNK)
    span = NS * CHUNK * NBUF
    e_pad = ((e + span - 1) // span) * span

    pad = e_pad - e
    if pad:
        pad_block = jnp.concatenate(
            [jnp.full((1, pad), n_pad - 1, jnp.int32),
             jnp.zeros((1, pad), jnp.int32)], axis=0)
        ei = jnp.concatenate([edge_index, pad_block], axis=1)
    else:
        ei = edge_index

    wcat = jnp.concatenate([weight, root_weight], axis=1)
    tlo, thi, root = _matmul(x, wcat, bias.reshape(1, -1), bn=1000)
    agg, deg = _sc_aggregate(tlo, thi, ei, n)
    return _finalize(agg, deg.reshape(-1, 1), root, bn=1000)


# speedup vs baseline: 1.4389x; 1.1462x over previous
import functools

import jax
import jax.numpy as jnp
from jax import lax
from jax.experimental import pallas as pl
from jax.experimental.pallas import tpu as pltpu
from jax.experimental.pallas import tpu_sc as plsc

NC = 2
NS = 16
LANES = 16
CHUNK = 128
HALF = 128


def _mm_body(x_ref, w_ref, b_ref, tlo_ref, thi_ref, root_ref):
    o = jnp.dot(x_ref[...], w_ref[...], preferred_element_type=jnp.float32)
    d = tlo_ref.shape[1]
    tlo_ref[...] = o[:, :d]
    thi_ref[...] = o[:, d:2 * d]
    root_ref[...] = o[:, 2 * d:] + b_ref[...]


def _matmul(x, wcat, bias_row, bn):
    n, d_in = x.shape
    d_out = bias_row.shape[1]
    grid = n // bn
    return pl.pallas_call(
        _mm_body,
        grid=(grid,),
        in_specs=[
            pl.BlockSpec((bn, d_in), lambda i: (i, 0)),
            pl.BlockSpec((d_in, 2 * d_out), lambda i: (0, 0)),
            pl.BlockSpec((1, d_out), lambda i: (0, 0)),
        ],
        out_specs=[
            pl.BlockSpec((bn, HALF), lambda i: (i, 0)),
            pl.BlockSpec((bn, HALF), lambda i: (i, 0)),
            pl.BlockSpec((bn, d_out), lambda i: (i, 0)),
        ],
        out_shape=[
            jax.ShapeDtypeStruct((n, HALF), jnp.float32),
            jax.ShapeDtypeStruct((n, HALF), jnp.float32),
            jax.ShapeDtypeStruct((n, d_out), jnp.float32),
        ],
    )(x, wcat, bias_row)


def _sc_aggregate(tlo, thi, edge_index, n_nodes, n_edges):
    n_pad = ((n_nodes + NS * CHUNK - 1) // (NS * CHUNK)) * (NS * CHUNK)
    rows_per_tile = n_pad // NS
    num_chunks = n_edges // CHUNK

    mesh = plsc.VectorSubcoreMesh(core_axis_name="c", subcore_axis_name="s")

    @functools.partial(
        pl.kernel,
        mesh=mesh,
        out_type=(
            jax.ShapeDtypeStruct((NC, n_pad, HALF), jnp.float32),
            jax.ShapeDtypeStruct((n_pad,), jnp.float32),
        ),
        scratch_types=[
            pltpu.VMEM_SHARED((n_pad, HALF), jnp.float32),
            pltpu.VMEM_SHARED((n_pad,), jnp.float32),
            pltpu.VMEM((1, CHUNK), jnp.int32),
            pltpu.VMEM((1, CHUNK), jnp.int32),
            pltpu.VMEM((CHUNK, HALF), jnp.float32),
            pltpu.VMEM((CHUNK,), jnp.float32),
            pltpu.SemaphoreType.DMA,
        ],
    )
    def agg_kernel(tlo_hbm, thi_hbm, edges_hbm, agg_hbm, deg_hbm,
                   agg_s, deg_s, ridx, cidx, msgs, ones, sem):
        c = lax.axis_index("c")
        t = lax.axis_index("s")
        r0 = t * rows_per_tile

        def zrow(r, _):
            def zcol(j, _):
                msgs[r, pl.ds(j * LANES, LANES)] = jnp.zeros(
                    (LANES,), jnp.float32)
                return 0
            return lax.fori_loop(0, HALF // LANES, zcol, 0)
        lax.fori_loop(0, CHUNK, zrow, 0)

        def zon(j, _):
            ones[pl.ds(j * LANES, LANES)] = jnp.zeros((LANES,), jnp.float32)
            return 0
        lax.fori_loop(0, CHUNK // LANES, zon, 0)

        for b in range(rows_per_tile // CHUNK):
            pltpu.sync_copy(msgs, agg_s.at[pl.ds(r0 + b * CHUNK, CHUNK)])
            pltpu.sync_copy(ones, deg_s.at[pl.ds(r0 + b * CHUNK, CHUNK)])

        def son(j, _):
            ones[pl.ds(j * LANES, LANES)] = jnp.ones((LANES,), jnp.float32)
            return 0
        lax.fori_loop(0, CHUNK // LANES, son, 0)

        plsc.subcore_barrier()

        nk = (num_chunks - t + NS - 1) // NS

        def ebody(k, _):
            base = (t + k * NS) * CHUNK
            pltpu.sync_copy(edges_hbm.at[0, pl.ds(base, CHUNK)], ridx.at[0])
            pltpu.sync_copy(edges_hbm.at[1, pl.ds(base, CHUNK)], cidx.at[0])

            @pl.when(c == 0)
            def _():
                pltpu.async_copy(tlo_hbm.at[cidx.at[0]], msgs, sem).wait()

            @pl.when(c == 1)
            def _():
                pltpu.async_copy(thi_hbm.at[cidx.at[0]], msgs, sem).wait()

            pltpu.sync_copy(msgs, agg_s.at[ridx.at[0]], add=True)
            pltpu.sync_copy(ones, deg_s.at[ridx.at[0]], add=True)
            return 0
        lax.fori_loop(0, nk, ebody, 0)

        plsc.subcore_barrier()

        pltpu.sync_copy(agg_s.at[pl.ds(r0, rows_per_tile)],
                        agg_hbm.at[c, pl.ds(r0, rows_per_tile)])

        @pl.when(c == 0)
        def _():
            pltpu.sync_copy(deg_s.at[pl.ds(r0, rows_per_tile)],
                            deg_hbm.at[pl.ds(r0, rows_per_tile)])

    return agg_kernel(tlo, thi, edge_index)


def _fin_body(agg_ref, deg_ref, root_ref, out_ref):
    d = jnp.maximum(deg_ref[...], 1.0)
    a = jnp.concatenate([agg_ref[0], agg_ref[1]], axis=-1)
    out_ref[...] = a / d + root_ref[...]


def _finalize(agg, deg_col, root, bn):
    n, d_out = root.shape
    grid = n // bn
    return pl.pallas_call(
        _fin_body,
        grid=(grid,),
        in_specs=[
            pl.BlockSpec((NC, bn, HALF), lambda i: (0, i, 0)),
            pl.BlockSpec((bn, 1), lambda i: (i, 0)),
            pl.BlockSpec((bn, d_out), lambda i: (i, 0)),
        ],
        out_specs=pl.BlockSpec((bn, d_out), lambda i: (i, 0)),
        out_shape=jax.ShapeDtypeStruct((n, d_out), jnp.float32),
    )(agg, deg_col, root)


def kernel(x, edge_index, weight, root_weight, bias):
    n, _ = x.shape
    e = edge_index.shape[1]
    wcat = jnp.concatenate([weight, root_weight], axis=1)
    tlo, thi, root = _matmul(x, wcat, bias.reshape(1, -1), bn=1000)
    agg, deg = _sc_aggregate(tlo, thi, edge_index, n, e)
    return _finalize(agg, deg.reshape(-1, 1), root, bn=1000)
